# baseline (speedup 1.0000x reference)
import functools

import jax
import jax.numpy as jnp
from jax import lax
from jax.experimental import pallas as pl
from jax.experimental.pallas import tpu as pltpu
from jax.experimental.pallas import tpu_sc as plsc

N = 10000
E = 160000
ND = 128
ED = 16
H = 16
B = 8
DEPTH = 3
STEPS = 3

NC = 2
NS = 16
NW = NC * NS
N_PAD = 10016
EPW = 5120
E_PAD = EPW * NW
CH = 128
NCH = EPW // CH
RPT = N_PAD // NS

_SC_PARAMS = pltpu.CompilerParams(use_tc_tiling_on_sc=False)
_TC_BIG_VMEM = pltpu.CompilerParams(vmem_limit_bytes=100 * 1024 * 1024)


def _sc_gather(table, idx3):
    mesh = plsc.VectorSubcoreMesh(core_axis_name="c", subcore_axis_name="s")

    @functools.partial(
        pl.kernel, mesh=mesh,
        out_type=jax.ShapeDtypeStruct((E_PAD, H), jnp.float32),
        scratch_types=[
            pltpu.VMEM((NCH, CH), jnp.int32),
            pltpu.VMEM((EPW, H), jnp.float32),
            pltpu.VMEM((RPT, H), jnp.float32),
            pltpu.VMEM_SHARED((N_PAD, H), jnp.float32),
            pltpu.SemaphoreType.DMA,
        ],
        compiler_params=_SC_PARAMS,
    )
    def k(table_hbm, idx_hbm, out_hbm, idx_v, rows_v, stage_v, tab_s, sem):
        cid = lax.axis_index("c")
        sid = lax.axis_index("s")
        wid = sid * NC + cid
        pltpu.sync_copy(table_hbm.at[pl.ds(sid * RPT, RPT)], stage_v)
        pltpu.sync_copy(stage_v, tab_s.at[pl.ds(sid * RPT, RPT)])
        pltpu.sync_copy(idx_hbm.at[wid], idx_v)
        plsc.subcore_barrier()

        GP = 8

        def grp(g, carry):
            for j in range(GP):
                pltpu.async_copy(
                    tab_s.at[idx_v.at[g * GP + j]],
                    rows_v.at[pl.ds((g * GP + j) * CH, CH)], sem)
            for j in range(GP):
                pltpu.make_async_copy(
                    tab_s.at[idx_v.at[g * GP + j]],
                    rows_v.at[pl.ds((g * GP + j) * CH, CH)], sem).wait()
            return carry

        lax.fori_loop(0, NCH // GP, grp, 0)
        pltpu.sync_copy(rows_v, out_hbm.at[pl.ds(wid * EPW, EPW)])

    return k(table, idx3)


def _sc_scatter(vals, idx3, zeros_np):
    mesh = plsc.VectorSubcoreMesh(core_axis_name="c", subcore_axis_name="s")

    @functools.partial(
        pl.kernel, mesh=mesh,
        out_type=jax.ShapeDtypeStruct((NC * N_PAD, H), jnp.float32),
        scratch_types=[
            pltpu.VMEM((NCH, CH), jnp.int32),
            pltpu.VMEM((EPW, H), jnp.float32),
            pltpu.VMEM((RPT, H), jnp.float32),
            pltpu.VMEM_SHARED((N_PAD, H), jnp.float32),
            pltpu.SemaphoreType.DMA,
        ],
        compiler_params=_SC_PARAMS,
    )
    def k(vals_hbm, idx_hbm, zero_hbm, out_hbm, idx_v, vals_v, row_v, shared, sem):
        cid = lax.axis_index("c")
        sid = lax.axis_index("s")
        wid = sid * NC + cid
        pltpu.sync_copy(zero_hbm.at[pl.ds(sid * RPT, RPT)], row_v)
        pltpu.sync_copy(row_v, shared.at[pl.ds(sid * RPT, RPT)])
        pltpu.sync_copy(idx_hbm.at[wid], idx_v)
        pltpu.sync_copy(vals_hbm.at[pl.ds(wid * EPW, EPW)], vals_v)
        plsc.subcore_barrier()

        def chunk(c, carry):
            pltpu.sync_copy(vals_v.at[pl.ds(c * CH, CH)],
                            shared.at[idx_v.at[c]], add=True)
            return carry

        lax.fori_loop(0, NCH, chunk, 0)
        plsc.subcore_barrier()
        pltpu.sync_copy(shared.at[pl.ds(sid * RPT, RPT)], row_v)
        pltpu.sync_copy(row_v, out_hbm.at[pl.ds(cid * N_PAD + sid * RPT, RPT)])

    return k(vals, idx3, zeros_np)


MB = 8192
NMB = E_PAD // MB
MR = MB // 8
NPK = N_PAD * H // 128


def _lanes(v, j, w):
    return v[:, j * w:(j + 1) * w]


def _tc_prep(x_pad, W0, b0):
    def body(x_ref, w_ref, b_ref, o_ref):
        o_ref[...] = jnp.maximum(
            jnp.dot(x_ref[...], w_ref[...],
                    preferred_element_type=jnp.float32) + b_ref[...], 0.0)

    return pl.pallas_call(
        body,
        out_shape=jax.ShapeDtypeStruct((N_PAD, H), jnp.float32),
    )(x_pad, W0, b0.reshape(1, H))


def _tc_msg(ea128, src128, Wnn1, bnn1, Wnn2, bnn2, Rm, Sm):
    def body(e_ref, s_ref, w1_ref, b1_ref, w2_ref, b2_ref, r_ref, sm_ref,
             o_ref):
        eb = e_ref[...]
        sb = s_ref[...]
        ecat = jnp.concatenate([_lanes(eb, j, H) for j in range(8)], axis=0)
        scat = jnp.concatenate([_lanes(sb, j, H) for j in range(8)], axis=0)
        z = jnp.maximum(
            jnp.dot(ecat, w1_ref[...],
                    preferred_element_type=jnp.float32) + b1_ref[...], 0.0)
        ew = jnp.dot(z.astype(jnp.bfloat16),
                     w2_ref[...].astype(jnp.bfloat16),
                     preferred_element_type=jnp.float32) + b2_ref[...]
        rep = jnp.dot(scat.astype(jnp.bfloat16),
                      r_ref[...].astype(jnp.bfloat16),
                      preferred_element_type=jnp.float32)
        pm = jnp.dot(ew * rep, sm_ref[...],
                     preferred_element_type=jnp.float32)
        o_ref[...] = jnp.concatenate(
            [pm[j * MR:(j + 1) * MR] for j in range(8)], axis=1)

    return pl.pallas_call(
        body,
        grid=(NMB,),
        in_specs=[
            pl.BlockSpec((MR, 128), lambda i: (i, 0)),
            pl.BlockSpec((MR, 128), lambda i: (i, 0)),
            pl.BlockSpec((ED, 2 * H), lambda i: (0, 0)),
            pl.BlockSpec((1, 2 * H), lambda i: (0, 0)),
            pl.BlockSpec((2 * H, H * H), lambda i: (0, 0)),
            pl.BlockSpec((1, H * H), lambda i: (0, 0)),
            pl.BlockSpec((H, H * H), lambda i: (0, 0)),
            pl.BlockSpec((H * H, H), lambda i: (0, 0)),
        ],
        out_specs=pl.BlockSpec((MR, 128), lambda i: (i, 0)),
        out_shape=jax.ShapeDtypeStruct((E_PAD * H // 128, 128), jnp.float32),
        compiler_params=_TC_BIG_VMEM,
    )(ea128, src128, Wnn1, bnn1.reshape(1, 2 * H),
      Wnn2, bnn2.reshape(1, H * H), Rm, Sm)


def _tc_update(agg128, deg128, h128, root, cb, WihT, WhhT, bih, bhh):
    def body(a_ref, d_ref, h_ref, root_ref, cb_ref,
             wi_ref, wh_ref, bi_ref, bh_ref, o_ref):
        av = a_ref[...]
        dv = d_ref[...]
        hv = h_ref[...]
        a0 = jnp.concatenate([_lanes(av[:NPK], j, H) for j in range(8)], 0)
        a1 = jnp.concatenate([_lanes(av[NPK:], j, H) for j in range(8)], 0)
        d0 = jnp.concatenate([_lanes(dv[:NPK], j, H) for j in range(8)], 0)
        d1 = jnp.concatenate([_lanes(dv[NPK:], j, H) for j in range(8)], 0)
        hj = jnp.concatenate([_lanes(hv, j, H) for j in range(8)], 0)
        dinv = 1.0 / jnp.maximum(d0 + d1, 1.0)
        m = jnp.maximum(
            (a0 + a1) * dinv
            + jnp.dot(hj, root_ref[...],
                      preferred_element_type=jnp.float32) + cb_ref[...],
            0.0)
        gi = jnp.dot(m, wi_ref[...],
                     preferred_element_type=jnp.float32) + bi_ref[...]
        gh = jnp.dot(hj, wh_ref[...],
                     preferred_element_type=jnp.float32) + bh_ref[...]
        r = jax.nn.sigmoid(gi[:, :H] + gh[:, :H])
        zg = jax.nn.sigmoid(gi[:, H:2 * H] + gh[:, H:2 * H])
        ng = jnp.tanh(gi[:, 2 * H:] + r * gh[:, 2 * H:])
        hn = (1.0 - zg) * ng + zg * hj
        o_ref[...] = jnp.concatenate(
            [hn[j * NPK:(j + 1) * NPK] for j in range(8)], axis=1)

    return pl.pallas_call(
        body,
        out_shape=jax.ShapeDtypeStruct((NPK, 128), jnp.float32),
        compiler_params=_TC_BIG_VMEM,
    )(agg128, deg128, h128, root, cb.reshape(1, H), WihT, WhhT,
      bih.reshape(1, 3 * H), bhh.reshape(1, 3 * H))


def _tc_final(h, batch2, lWihT, lWhhT, lbih, lbhh, W1, b1, W2, b2,
              W3, b3, W4, b4):
    NPG = N // B

    def body(h_ref, b_ref, wi_ref, wh_ref, bi_ref, bh_ref,
             w1_ref, b1_ref, w2_ref, b2_ref, w3_ref, b3_ref, w4_ref, b4_ref,
             pr_ref, val_ref):
        hv = h_ref[...]
        bvec = b_ref[...]
        seg = lax.broadcasted_iota(jnp.int32, (N_PAD, B), 1)
        M = (bvec == seg).astype(jnp.float32)
        valid = (bvec < B)

        q_star = jnp.zeros((B, 2 * H), jnp.float32)
        hs = jnp.zeros((B, H), jnp.float32)
        cs = jnp.zeros((B, H), jnp.float32)
        for _ in range(STEPS):
            g = (jnp.dot(q_star, wi_ref[...],
                         preferred_element_type=jnp.float32) + bi_ref[...]
                 + jnp.dot(hs, wh_ref[...],
                           preferred_element_type=jnp.float32) + bh_ref[...])
            i_g = jax.nn.sigmoid(g[:, :H])
            f_g = jax.nn.sigmoid(g[:, H:2 * H])
            g_g = jnp.tanh(g[:, 2 * H:3 * H])
            o_g = jax.nn.sigmoid(g[:, 3 * H:])
            cs = f_g * cs + i_g * g_g
            hs = o_g * jnp.tanh(cs)
            q = hs
            qn = jnp.dot(M, q, preferred_element_type=jnp.float32)
            e = jnp.sum(hv * qn, axis=1, keepdims=True)
            em8 = jnp.max(jnp.where(M > 0, e, -jnp.inf), axis=0, keepdims=True)
            em8 = jnp.where(jnp.isfinite(em8), em8, 0.0)
            emn = jnp.sum(M * em8, axis=1, keepdims=True)
            ex = jnp.exp(jnp.where(valid, e - emn, -1e30))
            den8 = jnp.sum(M * ex, axis=0, keepdims=True)
            denn = jnp.sum(M * den8, axis=1, keepdims=True)
            a = ex / jnp.maximum(denn, 1e-16)
            r_read = lax.dot_general(M * a, hv, (((0,), (0,)), ((), ())),
                                     preferred_element_type=jnp.float32)
            q_star = jnp.concatenate([q, r_read], axis=1)

        v1 = jnp.maximum(
            jnp.dot(q_star, w1_ref[...],
                    preferred_element_type=jnp.float32) + b1_ref[...], 0.0)
        val_ref[...] = jnp.dot(v1, w2_ref[...],
                               preferred_element_type=jnp.float32) + b2_ref[...]

        l1 = jnp.maximum(
            jnp.dot(hv, w3_ref[...],
                    preferred_element_type=jnp.float32) + b3_ref[...], 0.0)
        logits = jnp.dot(l1, w4_ref[...],
                         preferred_element_type=jnp.float32) + b4_ref[...]
        gid = lax.broadcasted_iota(jnp.int32, (N_PAD, 1), 0) // NPG
        G = (gid == seg).astype(jnp.float32)
        lm = jnp.max(logits, axis=1, keepdims=True)
        m8 = jnp.max(jnp.where(G > 0, lm, -jnp.inf), axis=0, keepdims=True)
        m8 = jnp.where(jnp.isfinite(m8), m8, 0.0)
        mn = jnp.sum(G * m8, axis=1, keepdims=True)
        ex2 = jnp.exp(jnp.where(gid < B, logits - mn, -1e30))
        srow = jnp.sum(ex2, axis=1, keepdims=True)
        s8 = jnp.sum(G * srow, axis=0, keepdims=True)
        sn = jnp.sum(G * s8, axis=1, keepdims=True)
        pr = ex2 / jnp.maximum(sn, 1e-30)
        pr_ref[...] = pr[:N, :]

    return pl.pallas_call(
        body,
        out_shape=(
            jax.ShapeDtypeStruct((N, 2), jnp.float32),
            jax.ShapeDtypeStruct((B, 1), jnp.float32),
        ),
        compiler_params=_TC_BIG_VMEM,
    )(h, batch2, lWihT, lWhhT, lbih.reshape(1, 4 * H), lbhh.reshape(1, 4 * H),
      W1, b1.reshape(1, H), W2, b2.reshape(1, 1),
      W3, b3.reshape(1, 36), W4, b4.reshape(1, 2))


def kernel(x, edge_index, edge_attr, batch, W0, b0, Wnn1, bnn1, Wnn2, bnn2,
           root, conv_bias, gru_Wih, gru_Whh, gru_bih, gru_bhh,
           lstm_Wih, lstm_Whh, lstm_bih, lstm_bhh,
           W1, b1, W2, b2, W3, b3, W4, b4):
    f32 = jnp.float32
    x_pad = jnp.pad(x, ((0, N_PAD - N), (0, 0)))
    src = jnp.pad(edge_index[0], (0, E_PAD - E)).reshape(NW, NCH, CH)
    dst = jnp.pad(edge_index[1], (0, E_PAD - E),
                  constant_values=N).reshape(NW, NCH, CH)
    batch2 = jnp.pad(batch, (0, N_PAD - N), constant_values=B).reshape(N_PAD, 1)
    zeros_np = jnp.zeros((N_PAD, H), f32)
    onesE = jnp.ones((E_PAD, H), f32)

    Rm = jnp.kron(jnp.eye(H, dtype=f32), jnp.ones((1, H), f32))
    Sm = jnp.kron(jnp.ones((H, 1), f32), jnp.eye(H, dtype=f32))

    def to_flat(a128, rows):
        return a128.reshape(rows, H)

    def to_packed(aflat):
        return aflat.reshape(-1, 128)

    ea128 = jnp.pad(edge_attr.reshape(E * ED // 128, 128),
                    ((0, (E_PAD - E) * ED // 128), (0, 0)))

    h0 = _tc_prep(x_pad, W0, b0)
    h128 = to_packed(h0)

    deg2 = _sc_scatter(onesE, dst, zeros_np)
    deg128 = to_packed(deg2)

    for _ in range(DEPTH):
        out_src = _sc_gather(to_flat(h128, N_PAD), src)
        msg128 = _tc_msg(ea128, to_packed(out_src), Wnn1, bnn1, Wnn2, bnn2,
                         Rm, Sm)
        agg2 = _sc_scatter(to_flat(msg128, E_PAD), dst, zeros_np)
        h128 = _tc_update(to_packed(agg2), deg128, h128,
                          root, conv_bias, gru_Wih.T, gru_Whh.T,
                          gru_bih, gru_bhh)

    pr, value = _tc_final(to_flat(h128, N_PAD), batch2, lstm_Wih.T,
                          lstm_Whh.T, lstm_bih, lstm_bhh,
                          W1, b1, W2, b2, W3, b3, W4, b4)
    probs = pr.reshape(B, -1)
    return probs, value


# speedup vs baseline: 1.0031x; 1.0031x over previous
import functools

import jax
import jax.numpy as jnp
from jax import lax
from jax.experimental import pallas as pl
from jax.experimental.pallas import tpu as pltpu
from jax.experimental.pallas import tpu_sc as plsc

N = 10000
E = 160000
ND = 128
ED = 16
H = 16
B = 8
DEPTH = 3
STEPS = 3

NC = 2
NS = 16
NW = NC * NS
N_PAD = 10016
EPW = 5120
E_PAD = EPW * NW
CH = 128
NCH = EPW // CH
RPT = N_PAD // NS

_SC_PARAMS = pltpu.CompilerParams(use_tc_tiling_on_sc=False)
_TC_BIG_VMEM = pltpu.CompilerParams(vmem_limit_bytes=100 * 1024 * 1024)


def _sc_gather(table, idx3):
    mesh = plsc.VectorSubcoreMesh(core_axis_name="c", subcore_axis_name="s")

    @functools.partial(
        pl.kernel, mesh=mesh,
        out_type=jax.ShapeDtypeStruct((E_PAD, H), jnp.float32),
        scratch_types=[
            pltpu.VMEM((NCH, CH), jnp.int32),
            pltpu.VMEM((EPW, H), jnp.float32),
            pltpu.VMEM((RPT, H), jnp.float32),
            pltpu.VMEM_SHARED((N_PAD, H), jnp.float32),
            pltpu.SemaphoreType.DMA,
        ],
        compiler_params=_SC_PARAMS,
    )
    def k(table_hbm, idx_hbm, out_hbm, idx_v, rows_v, stage_v, tab_s, sem):
        cid = lax.axis_index("c")
        sid = lax.axis_index("s")
        wid = sid * NC + cid
        pltpu.sync_copy(table_hbm.at[pl.ds(sid * RPT, RPT)], stage_v)
        pltpu.sync_copy(stage_v, tab_s.at[pl.ds(sid * RPT, RPT)])
        pltpu.sync_copy(idx_hbm.at[wid], idx_v)
        plsc.subcore_barrier()

        GP = 8

        def grp(g, carry):
            for j in range(GP):
                pltpu.async_copy(
                    tab_s.at[idx_v.at[g * GP + j]],
                    rows_v.at[pl.ds((g * GP + j) * CH, CH)], sem)
            for j in range(GP):
                pltpu.make_async_copy(
                    tab_s.at[idx_v.at[g * GP + j]],
                    rows_v.at[pl.ds((g * GP + j) * CH, CH)], sem).wait()
            return carry

        lax.fori_loop(0, NCH // GP, grp, 0)
        pltpu.sync_copy(rows_v, out_hbm.at[pl.ds(wid * EPW, EPW)])

    return k(table, idx3)


def _sc_scatter(vals, idx3, zeros_np):
    mesh = plsc.VectorSubcoreMesh(core_axis_name="c", subcore_axis_name="s")

    @functools.partial(
        pl.kernel, mesh=mesh,
        out_type=jax.ShapeDtypeStruct((NC * N_PAD, H), jnp.float32),
        scratch_types=[
            pltpu.VMEM((NCH, CH), jnp.int32),
            pltpu.VMEM((EPW, H), jnp.float32),
            pltpu.VMEM((RPT, H), jnp.float32),
            pltpu.VMEM_SHARED((N_PAD, H), jnp.float32),
            pltpu.SemaphoreType.DMA,
        ],
        compiler_params=_SC_PARAMS,
    )
    def k(vals_hbm, idx_hbm, zero_hbm, out_hbm, idx_v, vals_v, row_v, shared, sem):
        cid = lax.axis_index("c")
        sid = lax.axis_index("s")
        wid = sid * NC + cid
        pltpu.sync_copy(zero_hbm.at[pl.ds(sid * RPT, RPT)], row_v)
        pltpu.sync_copy(row_v, shared.at[pl.ds(sid * RPT, RPT)])
        pltpu.sync_copy(idx_hbm.at[wid], idx_v)
        pltpu.sync_copy(vals_hbm.at[pl.ds(wid * EPW, EPW)], vals_v)
        plsc.subcore_barrier()

        def chunk(c, carry):
            pltpu.sync_copy(vals_v.at[pl.ds(c * CH, CH)],
                            shared.at[idx_v.at[c]], add=True)
            return carry

        lax.fori_loop(0, NCH, chunk, 0)
        plsc.subcore_barrier()
        pltpu.sync_copy(shared.at[pl.ds(sid * RPT, RPT)], row_v)
        pltpu.sync_copy(row_v, out_hbm.at[pl.ds(cid * N_PAD + sid * RPT, RPT)])

    return k(vals, idx3, zeros_np)


MB = 8192
NMB = E_PAD // MB
MR = MB // 8
NPK = N_PAD * H // 128


def _lanes(v, j, w):
    return v[:, j * w:(j + 1) * w]


def _tc_prep(x_pad, W0, b0):
    def body(x_ref, w_ref, b_ref, o_ref):
        o_ref[...] = jnp.maximum(
            jnp.dot(x_ref[...], w_ref[...],
                    preferred_element_type=jnp.float32) + b_ref[...], 0.0)

    return pl.pallas_call(
        body,
        out_shape=jax.ShapeDtypeStruct((N_PAD, H), jnp.float32),
    )(x_pad, W0, b0.reshape(1, H))


def _tc_msg(ea128, src128, Wnn1, bnn1, Wnn2, bnn2, Rm, Sm):
    def body(e_ref, s_ref, w1_ref, b1_ref, w2_ref, b2_ref, r_ref, sm_ref,
             o_ref):
        eb = e_ref[...]
        sb = s_ref[...]
        ecat = jnp.concatenate([_lanes(eb, j, H) for j in range(8)], axis=0)
        scat = jnp.concatenate([_lanes(sb, j, H) for j in range(8)], axis=0)
        z = jnp.maximum(
            jnp.dot(ecat, w1_ref[...],
                    preferred_element_type=jnp.float32) + b1_ref[...], 0.0)
        ew = jnp.dot(z, w2_ref[...],
                     preferred_element_type=jnp.float32) + b2_ref[...]
        rep = jnp.dot(scat, r_ref[...], preferred_element_type=jnp.float32)
        pm = jnp.dot(ew * rep, sm_ref[...],
                     preferred_element_type=jnp.float32)
        o_ref[...] = jnp.concatenate(
            [pm[j * MR:(j + 1) * MR] for j in range(8)], axis=1)

    return pl.pallas_call(
        body,
        grid=(NMB,),
        in_specs=[
            pl.BlockSpec((MR, 128), lambda i: (i, 0)),
            pl.BlockSpec((MR, 128), lambda i: (i, 0)),
            pl.BlockSpec((ED, 2 * H), lambda i: (0, 0)),
            pl.BlockSpec((1, 2 * H), lambda i: (0, 0)),
            pl.BlockSpec((2 * H, H * H), lambda i: (0, 0)),
            pl.BlockSpec((1, H * H), lambda i: (0, 0)),
            pl.BlockSpec((H, H * H), lambda i: (0, 0)),
            pl.BlockSpec((H * H, H), lambda i: (0, 0)),
        ],
        out_specs=pl.BlockSpec((MR, 128), lambda i: (i, 0)),
        out_shape=jax.ShapeDtypeStruct((E_PAD * H // 128, 128), jnp.float32),
        compiler_params=_TC_BIG_VMEM,
    )(ea128, src128, Wnn1, bnn1.reshape(1, 2 * H),
      Wnn2, bnn2.reshape(1, H * H), Rm, Sm)


def _tc_update(agg128, deg128, h128, root, cb, WihT, WhhT, bih, bhh):
    def body(a_ref, d_ref, h_ref, root_ref, cb_ref,
             wi_ref, wh_ref, bi_ref, bh_ref, o_ref):
        av = a_ref[...]
        dv = d_ref[...]
        hv = h_ref[...]
        outs = []
        for j in range(8):
            a0 = _lanes(av[:NPK], j, H)
            a1 = _lanes(av[NPK:], j, H)
            d0 = _lanes(dv[:NPK], j, H)
            d1 = _lanes(dv[NPK:], j, H)
            hj = _lanes(hv, j, H)
            dinv = 1.0 / jnp.maximum(d0 + d1, 1.0)
            m = jnp.maximum(
                (a0 + a1) * dinv
                + jnp.dot(hj, root_ref[...],
                          preferred_element_type=jnp.float32) + cb_ref[...],
                0.0)
            gi = jnp.dot(m, wi_ref[...],
                         preferred_element_type=jnp.float32) + bi_ref[...]
            gh = jnp.dot(hj, wh_ref[...],
                         preferred_element_type=jnp.float32) + bh_ref[...]
            r = jax.nn.sigmoid(gi[:, :H] + gh[:, :H])
            zg = jax.nn.sigmoid(gi[:, H:2 * H] + gh[:, H:2 * H])
            ng = jnp.tanh(gi[:, 2 * H:] + r * gh[:, 2 * H:])
            outs.append((1.0 - zg) * ng + zg * hj)
        o_ref[...] = jnp.concatenate(outs, axis=1)

    return pl.pallas_call(
        body,
        out_shape=jax.ShapeDtypeStruct((NPK, 128), jnp.float32),
        compiler_params=_TC_BIG_VMEM,
    )(agg128, deg128, h128, root, cb.reshape(1, H), WihT, WhhT,
      bih.reshape(1, 3 * H), bhh.reshape(1, 3 * H))


def _tc_final(h, batch2, lWihT, lWhhT, lbih, lbhh, W1, b1, W2, b2,
              W3, b3, W4, b4):
    NPG = N // B

    def body(h_ref, b_ref, wi_ref, wh_ref, bi_ref, bh_ref,
             w1_ref, b1_ref, w2_ref, b2_ref, w3_ref, b3_ref, w4_ref, b4_ref,
             pr_ref, val_ref):
        hv = h_ref[...]
        bvec = b_ref[...]
        seg = lax.broadcasted_iota(jnp.int32, (N_PAD, B), 1)
        M = (bvec == seg).astype(jnp.float32)
        valid = (bvec < B)

        q_star = jnp.zeros((B, 2 * H), jnp.float32)
        hs = jnp.zeros((B, H), jnp.float32)
        cs = jnp.zeros((B, H), jnp.float32)
        for _ in range(STEPS):
            g = (jnp.dot(q_star, wi_ref[...],
                         preferred_element_type=jnp.float32) + bi_ref[...]
                 + jnp.dot(hs, wh_ref[...],
                           preferred_element_type=jnp.float32) + bh_ref[...])
            i_g = jax.nn.sigmoid(g[:, :H])
            f_g = jax.nn.sigmoid(g[:, H:2 * H])
            g_g = jnp.tanh(g[:, 2 * H:3 * H])
            o_g = jax.nn.sigmoid(g[:, 3 * H:])
            cs = f_g * cs + i_g * g_g
            hs = o_g * jnp.tanh(cs)
            q = hs
            qn = jnp.dot(M, q, preferred_element_type=jnp.float32)
            e = jnp.sum(hv * qn, axis=1, keepdims=True)
            em8 = jnp.max(jnp.where(M > 0, e, -jnp.inf), axis=0, keepdims=True)
            em8 = jnp.where(jnp.isfinite(em8), em8, 0.0)
            emn = jnp.sum(M * em8, axis=1, keepdims=True)
            ex = jnp.exp(jnp.where(valid, e - emn, -1e30))
            den8 = jnp.sum(M * ex, axis=0, keepdims=True)
            denn = jnp.sum(M * den8, axis=1, keepdims=True)
            a = ex / jnp.maximum(denn, 1e-16)
            r_read = lax.dot_general(M * a, hv, (((0,), (0,)), ((), ())),
                                     preferred_element_type=jnp.float32)
            q_star = jnp.concatenate([q, r_read], axis=1)

        v1 = jnp.maximum(
            jnp.dot(q_star, w1_ref[...],
                    preferred_element_type=jnp.float32) + b1_ref[...], 0.0)
        val_ref[...] = jnp.dot(v1, w2_ref[...],
                               preferred_element_type=jnp.float32) + b2_ref[...]

        l1 = jnp.maximum(
            jnp.dot(hv, w3_ref[...],
                    preferred_element_type=jnp.float32) + b3_ref[...], 0.0)
        logits = jnp.dot(l1, w4_ref[...],
                         preferred_element_type=jnp.float32) + b4_ref[...]
        gid = lax.broadcasted_iota(jnp.int32, (N_PAD, 1), 0) // NPG
        G = (gid == seg).astype(jnp.float32)
        lm = jnp.max(logits, axis=1, keepdims=True)
        m8 = jnp.max(jnp.where(G > 0, lm, -jnp.inf), axis=0, keepdims=True)
        m8 = jnp.where(jnp.isfinite(m8), m8, 0.0)
        mn = jnp.sum(G * m8, axis=1, keepdims=True)
        ex2 = jnp.exp(jnp.where(gid < B, logits - mn, -1e30))
        srow = jnp.sum(ex2, axis=1, keepdims=True)
        s8 = jnp.sum(G * srow, axis=0, keepdims=True)
        sn = jnp.sum(G * s8, axis=1, keepdims=True)
        pr = ex2 / jnp.maximum(sn, 1e-30)
        pr_ref[...] = pr[:N, :]

    return pl.pallas_call(
        body,
        out_shape=(
            jax.ShapeDtypeStruct((N, 2), jnp.float32),
            jax.ShapeDtypeStruct((B, 1), jnp.float32),
        ),
        compiler_params=_TC_BIG_VMEM,
    )(h, batch2, lWihT, lWhhT, lbih.reshape(1, 4 * H), lbhh.reshape(1, 4 * H),
      W1, b1.reshape(1, H), W2, b2.reshape(1, 1),
      W3, b3.reshape(1, 36), W4, b4.reshape(1, 2))


def kernel(x, edge_index, edge_attr, batch, W0, b0, Wnn1, bnn1, Wnn2, bnn2,
           root, conv_bias, gru_Wih, gru_Whh, gru_bih, gru_bhh,
           lstm_Wih, lstm_Whh, lstm_bih, lstm_bhh,
           W1, b1, W2, b2, W3, b3, W4, b4):
    f32 = jnp.float32
    x_pad = jnp.pad(x, ((0, N_PAD - N), (0, 0)))
    src = jnp.pad(edge_index[0], (0, E_PAD - E)).reshape(NW, NCH, CH)
    dst = jnp.pad(edge_index[1], (0, E_PAD - E),
                  constant_values=N).reshape(NW, NCH, CH)
    batch2 = jnp.pad(batch, (0, N_PAD - N), constant_values=B).reshape(N_PAD, 1)
    zeros_np = jnp.zeros((N_PAD, H), f32)
    onesE = jnp.ones((E_PAD, H), f32)

    Rm = jnp.kron(jnp.eye(H, dtype=f32), jnp.ones((1, H), f32))
    Sm = jnp.kron(jnp.ones((H, 1), f32), jnp.eye(H, dtype=f32))

    def to_flat(a128, rows):
        return a128.reshape(rows, H)

    def to_packed(aflat):
        return aflat.reshape(-1, 128)

    ea128 = jnp.pad(edge_attr.reshape(E * ED // 128, 128),
                    ((0, (E_PAD - E) * ED // 128), (0, 0)))

    h0 = _tc_prep(x_pad, W0, b0)
    h128 = to_packed(h0)

    deg2 = _sc_scatter(onesE, dst, zeros_np)
    deg128 = to_packed(deg2)

    for _ in range(DEPTH):
        out_src = _sc_gather(to_flat(h128, N_PAD), src)
        msg128 = _tc_msg(ea128, to_packed(out_src), Wnn1, bnn1, Wnn2, bnn2,
                         Rm, Sm)
        agg2 = _sc_scatter(to_flat(msg128, E_PAD), dst, zeros_np)
        h128 = _tc_update(to_packed(agg2), deg128, h128,
                          root, conv_bias, gru_Wih.T, gru_Whh.T,
                          gru_bih, gru_bhh)

    pr, value = _tc_final(to_flat(h128, N_PAD), batch2, lstm_Wih.T,
                          lstm_Whh.T, lstm_bih, lstm_bhh,
                          W1, b1, W2, b2, W3, b3, W4, b4)
    probs = pr.reshape(B, -1)
    return probs, value
